# baseline (speedup 1.0000x reference)
import functools

import jax
import jax.numpy as jnp
from jax import lax
from jax.experimental import pallas as pl
from jax.experimental.pallas import tpu as pltpu
from jax.experimental.pallas import tpu_sc as plsc

_NC = 2
_NS = 16
_NW = _NC * _NS
_L = 16


def _worker_id():
    return lax.axis_index("s") * _NC + lax.axis_index("c")


def _splat(x):
    return jnp.broadcast_to(x, (_L,))


@functools.lru_cache(maxsize=None)
def _build_transpose(V, D):
    assert D == 64 and V % 2 == 0
    full_tiles = V // 128
    tail_w = V - full_tiles * 128
    per_w = (full_tiles + _NW - 1) // _NW
    nbuf = 4
    slots = ((per_w + nbuf - 1) // nbuf) * nbuf

    mesh = plsc.VectorSubcoreMesh(core_axis_name="c", subcore_axis_name="s")

    @functools.partial(
        pl.kernel,
        out_type=jax.ShapeDtypeStruct((V // 2, 2 * D), jnp.float32),
        mesh=mesh,
        compiler_params=pltpu.CompilerParams(
            use_tc_tiling_on_sc=True, needs_layout_passes=False),
        scratch_types=[
            *[pltpu.VMEM((D, 128), jnp.float32)] * nbuf,
            *[pltpu.VMEM((64, 128), jnp.float32)] * nbuf,
            *[pltpu.SemaphoreType.DMA] * nbuf,
            *[pltpu.SemaphoreType.DMA] * nbuf,
        ],
    )
    def ktr(tokT_hbm, tail2_hbm, out_hbm, *bufs):
        slab = bufs[:nbuf]
        trans = bufs[nbuf:2 * nbuf]
        lsem = bufs[2 * nbuf:3 * nbuf]
        ssem = bufs[3 * nbuf:]
        wid = _worker_id()

        lane = lax.iota(jnp.int32, _L)
        brow, bcol = [], []
        for g in range(8):
            p = lane + g * _L
            brow.append(p % D)
            bcol.append(p // D)

        def vt_of(i):
            return i * _NW + wid

        def load(i, b, start):
            vt = vt_of(i)

            @pl.when(vt < full_tiles)
            def _():
                d = pltpu.make_async_copy(
                    tokT_hbm.at[:, pl.ds(vt * 128, 128)], slab[b], lsem[b])
                d.start() if start else d.wait()

        def store(i, b, start):
            vt = vt_of(i)

            @pl.when(vt < full_tiles)
            def _():
                d = pltpu.make_async_copy(
                    trans[b], out_hbm.at[pl.ds(vt * 64, 64), :], ssem[b])
                d.start() if start else d.wait()

        if tail_w:
            @pl.when(wid == 0)
            def _():
                pltpu.sync_copy(
                    tail2_hbm, out_hbm.at[pl.ds(full_tiles * 64, tail_w // 2)])

        for b in range(nbuf):
            load(b, b, start=True)

        def outer(o, carry):
            for b in range(nbuf):
                i = o * nbuf + b

                @pl.when((i >= nbuf) & (vt_of(i - nbuf) < full_tiles))
                def _():
                    store(i - nbuf, b, start=False)

                @pl.when(vt_of(i) < full_tiles)
                def _():
                    load(i, b, start=False)

                    @plsc.parallel_loop(0, 64, unroll=4)
                    def _(r):
                        for g in range(8):
                            vals = plsc.load_gather(
                                slab[b], [brow[g], bcol[g] + 2 * r])
                            trans[b][r, pl.ds(g * _L, _L)] = vals

                    store(i, b, start=True)

                    @pl.when(vt_of(i + nbuf) < full_tiles)
                    def _():
                        load(i + nbuf, b, start=True)
            return carry

        lax.fori_loop(0, slots // nbuf, outer, 0)
        for i in range(slots - nbuf, slots):
            b = i % nbuf

            @pl.when(vt_of(i) < full_tiles)
            def _():
                store(i, b, start=False)

    return ktr


@functools.lru_cache(maxsize=None)
def _build_gather(B, T, V, D):
    assert D == 64 and B % 128 == 0
    nbt = B // 128
    t_per_w = T * nbt // _NW
    assert t_per_w * _NW == T * nbt and t_per_w % 2 == 0
    nbuf = 2

    mesh = plsc.VectorSubcoreMesh(core_axis_name="c", subcore_axis_name="s")

    @functools.partial(
        pl.kernel,
        out_type=jax.ShapeDtypeStruct((T, D // 8, nbt, 8, 128), jnp.float32),
        mesh=mesh,
        compiler_params=pltpu.CompilerParams(
            use_tc_tiling_on_sc=True, needs_layout_passes=False),
        scratch_types=[
            pltpu.VMEM((T, D), jnp.float32),
            *[pltpu.VMEM((128,), jnp.int32)] * nbuf,
            *[pltpu.VMEM((128,), jnp.int32)] * nbuf,
            *[pltpu.VMEM((128, 2 * D), jnp.float32)] * nbuf,
            *[pltpu.VMEM((D // 8, 8, 128), jnp.float32)] * nbuf,
            *[pltpu.SemaphoreType.DMA] * nbuf,
            *[pltpu.SemaphoreType.DMA] * nbuf,
            *[pltpu.SemaphoreType.DMA] * nbuf,
        ],
    )
    def kg(xT_hbm, tok2_hbm, pos_hbm, out_hbm, pos_v, *bufs):
        idx = bufs[:nbuf]
        idx2 = bufs[nbuf:2 * nbuf]
        rows = bufs[2 * nbuf:3 * nbuf]
        trans = bufs[3 * nbuf:4 * nbuf]
        isem = bufs[4 * nbuf:5 * nbuf]
        gsem = bufs[5 * nbuf:6 * nbuf]
        osem = bufs[6 * nbuf:]
        wid = _worker_id()
        bt = wid % nbt
        t0 = (wid // nbt) * t_per_w
        pltpu.sync_copy(pos_hbm, pos_v)
        lane = lax.iota(jnp.int32, _L)

        def idx_desc(i, b):
            return pltpu.make_async_copy(
                xT_hbm.at[t0 + i, pl.ds(bt * 128, 128)], idx[b], isem[b])

        def gather_desc(b):
            return pltpu.make_async_copy(
                tok2_hbm.at[idx2[b]], rows[b], gsem[b])

        def out_desc(i, b):
            return pltpu.make_async_copy(
                trans[b], out_hbm.at[t0 + i, :, bt, :, :], osem[b])

        idx_desc(0, 0).start()

        def step(i, b):
            idx_desc(i, b).wait()
            for g in range(8):
                sl = pl.ds(g * _L, _L)
                idx2[b][sl] = lax.shift_right_logical(idx[b][sl], 1)

            @pl.when(i + 1 < t_per_w)
            def _():
                idx_desc(i + 1, 1 - b).start()

            gather_desc(b).start()

            prow, pcol = [], []
            for g in range(8):
                sl = pl.ds(g * _L, _L)
                prow.append(lane + g * _L)
                pcol.append((idx[b][sl] & 1) * D)

            @pl.when(i >= nbuf)
            def _():
                out_desc(i - nbuf, b).wait()

            gather_desc(b).wait()
            t = t0 + i

            @plsc.parallel_loop(0, D, unroll=4)
            def _(d):
                padd = plsc.load_gather(pos_v, [_splat(t), _splat(d)])
                for g in range(8):
                    vals = plsc.load_gather(rows[b], [prow[g], pcol[g] + d])
                    trans[b][d // 8, d % 8, pl.ds(g * _L, _L)] = vals + padd

            out_desc(i, b).start()

        def outer(o, carry):
            for b in range(nbuf):
                step(o * nbuf + b, b)
            return carry

        lax.fori_loop(0, t_per_w // nbuf, outer, 0)
        for i in range(t_per_w - nbuf, t_per_w):
            out_desc(i, i % nbuf).wait()

    return kg


def kernel(x, token_table, pos_table):
    B, T = x.shape
    V, D = token_table.shape
    full = (V // 128) * 128
    tail2 = token_table[full:].reshape((V - full) // 2, 2 * D)
    tok2 = _build_transpose(V, D)(token_table.T, tail2)
    out5 = _build_gather(B, T, V, D)(x.T, tok2, pos_table)
    return out5.transpose(2, 4, 0, 1, 3).reshape(B, T, D)
